# Initial kernel scaffold; baseline (speedup 1.0000x reference)
import jax
import jax.numpy as jnp
from jax.experimental import pallas as pl


def kernel(x, edge_index, W1, b1, W2, b2, W3, b3, W4, b4):
    raise NotImplementedError("write your pallas kernel here")


# speedup vs baseline: 5.9867x; 5.9867x over previous
import functools

import jax
import jax.numpy as jnp
from jax import lax
from jax.experimental import pallas as pl
from jax.experimental.pallas import tpu as pltpu
from jax.experimental.pallas import tpu_sc as plsc

F32 = jnp.float32

NC, NS, L = 2, 16, 16
NW = NC * NS
C = 128
H = 128


def _zero_rows(ref, nrows):
    z = jnp.zeros((16,), F32)

    def body(r, _):
        for k in range(H // 16):
            ref[r, pl.ds(k * 16, 16)] = z
        return 0

    lax.fori_loop(0, nrows, body, 0)


def _make_hist(NPAD, EP):
    chunks = EP // (NW * C)
    rows_t = NPAD // NS
    hop = rows_t // 4
    mesh = plsc.VectorSubcoreMesh(core_axis_name="c", subcore_axis_name="s")

    @functools.partial(
        pl.kernel,
        out_type=jax.ShapeDtypeStruct((NC, NPAD, H), F32),
        mesh=mesh,
        scratch_types=[
            pltpu.VMEM((C,), jnp.int32),
            pltpu.VMEM((C, H), F32),
            pltpu.VMEM((hop, H), F32),
            pltpu.VMEM_SHARED((NPAD, H), F32),
        ],
    )
    def hist(dst_hbm, out_hbm, idx_v, ones_v, stage_v, deg_sh):
        cid = lax.axis_index("c")
        sid = lax.axis_index("s")
        wid = cid * NS + sid
        one = jnp.ones((16,), F32)

        def fill_ones(r, _):
            for k in range(H // 16):
                ones_v[r, pl.ds(k * 16, 16)] = one
            return 0

        lax.fori_loop(0, C, fill_ones, 0)
        _zero_rows(stage_v, hop)
        for p in range(4):
            pltpu.sync_copy(stage_v,
                            deg_sh.at[pl.ds(sid * rows_t + p * hop, hop)])
        plsc.subcore_barrier()

        def chunk(ch, _):
            base = wid * (EP // NW) + ch * C
            pltpu.sync_copy(dst_hbm.at[pl.ds(base, C)], idx_v)
            pltpu.sync_copy(ones_v, deg_sh.at[idx_v], add=True)
            return 0

        lax.fori_loop(0, chunks, chunk, 0)
        plsc.subcore_barrier()
        for p in range(4):
            pltpu.sync_copy(deg_sh.at[pl.ds(sid * rows_t + p * hop, hop)],
                            stage_v)
            pltpu.sync_copy(
                stage_v, out_hbm.at[cid, pl.ds(sid * rows_t + p * hop, hop)])

    return hist


def _make_scatter(NPAD, EP):
    chunks = EP // (NW * C)
    rows_t = NPAD // NS
    hop = rows_t // 4
    mesh = plsc.VectorSubcoreMesh(core_axis_name="c", subcore_axis_name="s")

    @functools.partial(
        pl.kernel,
        out_type=jax.ShapeDtypeStruct((NC, NPAD, H), F32),
        mesh=mesh,
        scratch_types=[
            pltpu.VMEM((C,), jnp.int32),
            pltpu.VMEM((C,), jnp.int32),
            pltpu.VMEM((C, H), F32),
            pltpu.VMEM((hop, H), F32),
            pltpu.VMEM_SHARED((NPAD, H), F32),
            pltpu.SemaphoreType.DMA,
        ],
    )
    def scatter(ht_hbm, src_hbm, dst_hbm, out_hbm, isrc_v, idst_v, rows_v,
                stage_v, acc_sh, sem):
        cid = lax.axis_index("c")
        sid = lax.axis_index("s")
        wid = cid * NS + sid

        _zero_rows(stage_v, hop)
        for p in range(4):
            pltpu.sync_copy(stage_v,
                            acc_sh.at[pl.ds(sid * rows_t + p * hop, hop)])
        plsc.subcore_barrier()

        def chunk(ch, _):
            base = wid * (EP // NW) + ch * C
            pltpu.sync_copy(src_hbm.at[pl.ds(base, C)], isrc_v)
            pltpu.sync_copy(dst_hbm.at[pl.ds(base, C)], idst_v)
            pltpu.async_copy(ht_hbm.at[isrc_v], rows_v, sem).wait()
            pltpu.sync_copy(rows_v, acc_sh.at[idst_v], add=True)
            return 0

        lax.fori_loop(0, chunks, chunk, 0)
        plsc.subcore_barrier()
        for p in range(4):
            pltpu.sync_copy(acc_sh.at[pl.ds(sid * rows_t + p * hop, hop)],
                            stage_v)
            pltpu.sync_copy(
                stage_v, out_hbm.at[cid, pl.ds(sid * rows_t + p * hop, hop)])

    return scatter


def _make_edge_score(NPAD, EP):
    chunks = EP // (NW * C)
    mesh = plsc.VectorSubcoreMesh(core_axis_name="c", subcore_axis_name="s")

    @functools.partial(
        pl.kernel,
        out_type=jax.ShapeDtypeStruct((EP,), F32),
        mesh=mesh,
        scratch_types=[
            pltpu.VMEM((C,), jnp.int32),
            pltpu.VMEM((C,), jnp.int32),
            pltpu.VMEM((C, H), F32),
            pltpu.VMEM((C, H), F32),
            pltpu.VMEM((C,), F32),
            pltpu.VMEM((H,), F32),
            pltpu.VMEM((16,), F32),
            pltpu.VMEM((16,), F32),
            pltpu.SemaphoreType.DMA,
            pltpu.SemaphoreType.DMA,
        ],
    )
    def edge_score(a_hbm, b_hbm, src_hbm, dst_hbm, w4_hbm, b4_hbm, out_hbm,
                   isrc_v, idst_v, rows_a, rows_b, sout_v, w4_v, b4_v, tmp_v,
                   sema, semb):
        cid = lax.axis_index("c")
        sid = lax.axis_index("s")
        wid = cid * NS + sid
        pltpu.sync_copy(w4_hbm, w4_v)
        pltpu.sync_copy(b4_hbm, b4_v)
        zero16 = jnp.zeros((16,), F32)
        b4s = b4_v[...][0]
        lane = lax.iota(jnp.int32, 16)

        def chunk(ch, _):
            base = wid * (EP // NW) + ch * C
            pltpu.sync_copy(src_hbm.at[pl.ds(base, C)], isrc_v)
            pltpu.sync_copy(dst_hbm.at[pl.ds(base, C)], idst_v)
            da = pltpu.async_copy(a_hbm.at[isrc_v], rows_a, sema)
            db = pltpu.async_copy(b_hbm.at[idst_v], rows_b, semb)
            da.wait()
            db.wait()

            def edge(e, outvec):
                acc = zero16
                for k in range(H // 16):
                    s = pl.ds(k * 16, 16)
                    v = rows_a[e, s] + rows_b[e, s]
                    acc = acc + jnp.maximum(v, 0.0) * w4_v[s]
                tot = b4s
                for i in range(16):
                    tot = tot + acc[i]
                outvec = jnp.where(lane == (e & 15), tot, outvec)
                flush = (e & 15) == 15

                @pl.when(flush)
                def _():
                    off = pl.multiple_of(e - 15, 16)
                    sout_v[pl.ds(off, 16)] = outvec

                return jnp.where(flush, zero16, outvec)

            lax.fori_loop(0, C, edge, zero16)
            pltpu.sync_copy(sout_v, out_hbm.at[pl.ds(base, C)])
            return 0

        lax.fori_loop(0, chunks, chunk, 0)

    return edge_score


RB = 1024


def _dinv_of(dp0, dp1):
    deg = dp0[:, :1] + dp1[:, :1] + 1.0
    return lax.rsqrt(deg)


def _mm_t(x, w):
    return lax.dot_general(x, w, (((1,), (1,)), ((), ())),
                           preferred_element_type=F32)


def _tc1_body(dp_ref, x_ref, w1_ref, ht_ref):
    dinv = _dinv_of(dp_ref[0], dp_ref[1])
    ht_ref[...] = _mm_t(x_ref[...], w1_ref[...]) * dinv


def _tc2_body(dp_ref, acc_ref, ht_ref, w2_ref, b1_ref, out_ref):
    dinv = _dinv_of(dp_ref[0], dp_ref[1])
    h1 = (acc_ref[0] + acc_ref[1] + ht_ref[...]) * dinv + b1_ref[...]
    h1 = jnp.maximum(h1, 0.0)
    out_ref[...] = _mm_t(h1, w2_ref[...]) * dinv


def _tc3_body(dp_ref, acc_ref, ht_ref, w3a_ref, w3b_ref, b2_ref, b3_ref,
              a_ref, b_ref):
    dinv = _dinv_of(dp_ref[0], dp_ref[1])
    h2 = (acc_ref[0] + acc_ref[1] + ht_ref[...]) * dinv + b2_ref[...]
    a_ref[...] = _mm_t(h2, w3a_ref[...]) + b3_ref[...]
    b_ref[...] = _mm_t(h2, w3b_ref[...])


def kernel(x, edge_index, W1, b1, W2, b2, W3, b3, W4, b4):
    N = x.shape[0]
    E = edge_index.shape[1]
    NPAD = -(-N // (NS * 16)) * (NS * 16)
    EP = -(-E // (NW * C)) * (NW * C)

    src = edge_index[0]
    dst = edge_index[1]
    if EP != E:
        pad_s = jnp.zeros((EP - E,), jnp.int32)
        pad_d = jnp.full((EP - E,), N, jnp.int32)
        src = jnp.concatenate([src, pad_s])
        dst = jnp.concatenate([dst, pad_d])
    xp = jnp.pad(x, ((0, NPAD - N), (0, 0)))

    w3a = W3[:, :H]
    w3b = W3[:, H:]
    b1r = b1.reshape(1, H)
    b2r = b2.reshape(1, H)
    b3r = b3.reshape(1, H)
    w4 = W4.reshape(H)

    hist = _make_hist(NPAD, EP)
    scatter = _make_scatter(NPAD, EP)
    edge_score = _make_edge_score(NPAD, EP)

    deg_parts = hist(dst)

    grid = NPAD // RB
    dp_s = pl.BlockSpec((NC, RB, H), lambda i: (0, i, 0))
    acc_s = pl.BlockSpec((NC, RB, H), lambda i: (0, i, 0))
    rows_s = pl.BlockSpec((RB, H), lambda i: (i, 0))
    w_s = pl.BlockSpec((H, H), lambda i: (0, 0))
    b_s = pl.BlockSpec((1, H), lambda i: (0, 0))

    ht1 = pl.pallas_call(
        _tc1_body,
        grid=(grid,),
        in_specs=[dp_s, rows_s, w_s],
        out_specs=rows_s,
        out_shape=jax.ShapeDtypeStruct((NPAD, H), F32),
    )(deg_parts, xp, W1)

    acc1 = scatter(ht1, src, dst)

    ht2 = pl.pallas_call(
        _tc2_body,
        grid=(grid,),
        in_specs=[dp_s, acc_s, rows_s, w_s, b_s],
        out_specs=rows_s,
        out_shape=jax.ShapeDtypeStruct((NPAD, H), F32),
    )(deg_parts, acc1, ht1, W2, b1r)

    acc2 = scatter(ht2, src, dst)

    A, B = pl.pallas_call(
        _tc3_body,
        grid=(grid,),
        in_specs=[dp_s, acc_s, rows_s, w_s, w_s, b_s, b_s],
        out_specs=[rows_s, rows_s],
        out_shape=[
            jax.ShapeDtypeStruct((NPAD, H), F32),
            jax.ShapeDtypeStruct((NPAD, H), F32),
        ],
    )(deg_parts, acc2, ht2, w3a, w3b, b2r, b3r)

    b4v = jnp.full((16,), b4[0], F32)
    out = edge_score(A, B, src, dst, w4, b4v)

    return out[:E]


# speedup vs baseline: 6.5046x; 1.0865x over previous
import functools

import jax
import jax.numpy as jnp
from jax import lax
from jax.experimental import pallas as pl
from jax.experimental.pallas import tpu as pltpu
from jax.experimental.pallas import tpu_sc as plsc

F32 = jnp.float32

NC, NS, L = 2, 16, 16
NW = NC * NS
C = 128
H = 128


def _zero_rows(ref, nrows):
    z = jnp.zeros((16,), F32)

    def body(r, _):
        for k in range(H // 16):
            ref[r, pl.ds(k * 16, 16)] = z
        return 0

    lax.fori_loop(0, nrows, body, 0)


def _make_hist(NPAD, EP):
    chunks = EP // (NW * C)
    rows_t = NPAD // NS
    hop = rows_t // 4
    mesh = plsc.VectorSubcoreMesh(core_axis_name="c", subcore_axis_name="s")

    @functools.partial(
        pl.kernel,
        out_type=jax.ShapeDtypeStruct((NC, NPAD, H), F32),
        mesh=mesh,
        scratch_types=[
            pltpu.VMEM((C,), jnp.int32),
            pltpu.VMEM((C, H), F32),
            pltpu.VMEM((hop, H), F32),
            pltpu.VMEM_SHARED((NPAD, H), F32),
        ],
    )
    def hist(dst_hbm, out_hbm, idx_v, ones_v, stage_v, deg_sh):
        cid = lax.axis_index("c")
        sid = lax.axis_index("s")
        wid = cid * NS + sid
        one = jnp.ones((16,), F32)

        def fill_ones(r, _):
            for k in range(H // 16):
                ones_v[r, pl.ds(k * 16, 16)] = one
            return 0

        lax.fori_loop(0, C, fill_ones, 0)
        _zero_rows(stage_v, hop)
        for p in range(4):
            pltpu.sync_copy(stage_v,
                            deg_sh.at[pl.ds(sid * rows_t + p * hop, hop)])
        plsc.subcore_barrier()

        def chunk(ch, _):
            base = wid * (EP // NW) + ch * C
            pltpu.sync_copy(dst_hbm.at[pl.ds(base, C)], idx_v)
            pltpu.sync_copy(ones_v, deg_sh.at[idx_v], add=True)
            return 0

        lax.fori_loop(0, chunks, chunk, 0)
        plsc.subcore_barrier()
        for p in range(4):
            pltpu.sync_copy(deg_sh.at[pl.ds(sid * rows_t + p * hop, hop)],
                            stage_v)
            pltpu.sync_copy(
                stage_v, out_hbm.at[cid, pl.ds(sid * rows_t + p * hop, hop)])

    return hist


def _make_scatter(NPAD, EP):
    chunks = EP // (NW * C)
    rows_t = NPAD // NS
    hop = rows_t // 4
    mesh = plsc.VectorSubcoreMesh(core_axis_name="c", subcore_axis_name="s")

    @functools.partial(
        pl.kernel,
        out_type=jax.ShapeDtypeStruct((NC, NPAD, H), F32),
        mesh=mesh,
        scratch_types=[
            pltpu.VMEM((C,), jnp.int32),
            pltpu.VMEM((C,), jnp.int32),
            pltpu.VMEM((C, H), F32),
            pltpu.VMEM((hop, H), F32),
            pltpu.VMEM_SHARED((NPAD, H), F32),
            pltpu.SemaphoreType.DMA,
        ],
    )
    def scatter(ht_hbm, src_hbm, dst_hbm, out_hbm, isrc_v, idst_v, rows_v,
                stage_v, acc_sh, sem):
        cid = lax.axis_index("c")
        sid = lax.axis_index("s")
        wid = cid * NS + sid

        _zero_rows(stage_v, hop)
        for p in range(4):
            pltpu.sync_copy(stage_v,
                            acc_sh.at[pl.ds(sid * rows_t + p * hop, hop)])
        plsc.subcore_barrier()

        def chunk(ch, _):
            base = wid * (EP // NW) + ch * C
            pltpu.sync_copy(src_hbm.at[pl.ds(base, C)], isrc_v)
            pltpu.sync_copy(dst_hbm.at[pl.ds(base, C)], idst_v)
            pltpu.async_copy(ht_hbm.at[isrc_v], rows_v, sem).wait()
            pltpu.sync_copy(rows_v, acc_sh.at[idst_v], add=True)
            return 0

        lax.fori_loop(0, chunks, chunk, 0)
        plsc.subcore_barrier()
        for p in range(4):
            pltpu.sync_copy(acc_sh.at[pl.ds(sid * rows_t + p * hop, hop)],
                            stage_v)
            pltpu.sync_copy(
                stage_v, out_hbm.at[cid, pl.ds(sid * rows_t + p * hop, hop)])

    return scatter


def _make_edge_score(NPAD, EP):
    chunks = EP // (NW * C)
    mesh = plsc.VectorSubcoreMesh(core_axis_name="c", subcore_axis_name="s")

    @functools.partial(
        pl.kernel,
        out_type=jax.ShapeDtypeStruct((EP, 16), F32),
        mesh=mesh,
        scratch_types=[
            pltpu.VMEM((C,), jnp.int32),
            pltpu.VMEM((C,), jnp.int32),
            pltpu.VMEM((C, H), F32),
            pltpu.VMEM((C, H), F32),
            pltpu.VMEM((C, 16), F32),
            pltpu.VMEM((H,), F32),
            pltpu.SemaphoreType.DMA,
            pltpu.SemaphoreType.DMA,
        ],
    )
    def edge_score(a_hbm, b_hbm, src_hbm, dst_hbm, w4_hbm, out_hbm,
                   isrc_v, idst_v, rows_a, rows_b, s16_v, w4_v, sema, semb):
        cid = lax.axis_index("c")
        sid = lax.axis_index("s")
        wid = cid * NS + sid
        pltpu.sync_copy(w4_hbm, w4_v)
        zero16 = jnp.zeros((16,), F32)

        def chunk(ch, _):
            base = wid * (EP // NW) + ch * C
            pltpu.sync_copy(src_hbm.at[pl.ds(base, C)], isrc_v)
            pltpu.sync_copy(dst_hbm.at[pl.ds(base, C)], idst_v)
            da = pltpu.async_copy(a_hbm.at[isrc_v], rows_a, sema)
            db = pltpu.async_copy(b_hbm.at[idst_v], rows_b, semb)
            da.wait()
            db.wait()

            def edge(e, _):
                acc = zero16
                for k in range(H // 16):
                    s = pl.ds(k * 16, 16)
                    v = rows_a[e, s] + rows_b[e, s]
                    acc = acc + jnp.maximum(v, 0.0) * w4_v[s]
                s16_v[e, :] = acc
                return 0

            lax.fori_loop(0, C, edge, 0)
            pltpu.sync_copy(s16_v, out_hbm.at[pl.ds(base, C)])
            return 0

        lax.fori_loop(0, chunks, chunk, 0)

    return edge_score


RB = 1024


def _dinv_of(dp0, dp1):
    deg = dp0[:, :1] + dp1[:, :1] + 1.0
    return lax.rsqrt(deg)


def _mm_t(x, w):
    return lax.dot_general(x, w, (((1,), (1,)), ((), ())),
                           preferred_element_type=F32)


def _tc1_body(dp_ref, x_ref, w1_ref, ht_ref):
    dinv = _dinv_of(dp_ref[0], dp_ref[1])
    ht_ref[...] = _mm_t(x_ref[...], w1_ref[...]) * dinv


def _tc2_body(dp_ref, acc_ref, ht_ref, w2_ref, b1_ref, out_ref):
    dinv = _dinv_of(dp_ref[0], dp_ref[1])
    h1 = (acc_ref[0] + acc_ref[1] + ht_ref[...]) * dinv + b1_ref[...]
    h1 = jnp.maximum(h1, 0.0)
    out_ref[...] = _mm_t(h1, w2_ref[...]) * dinv


def _tc3_body(dp_ref, acc_ref, ht_ref, w3a_ref, w3b_ref, b2_ref, b3_ref,
              a_ref, b_ref):
    dinv = _dinv_of(dp_ref[0], dp_ref[1])
    h2 = (acc_ref[0] + acc_ref[1] + ht_ref[...]) * dinv + b2_ref[...]
    a_ref[...] = _mm_t(h2, w3a_ref[...]) + b3_ref[...]
    b_ref[...] = _mm_t(h2, w3b_ref[...])


def _tc4_body(s_ref, b4_ref, o_ref):
    ones_row = jnp.ones((1, 16), F32)
    for j in range(s_ref.shape[0]):
        o_ref[j, :] = (_mm_t(ones_row, s_ref[j]) + b4_ref[0])[0]


def kernel(x, edge_index, W1, b1, W2, b2, W3, b3, W4, b4):
    N = x.shape[0]
    E = edge_index.shape[1]
    NPAD = -(-N // (NS * 16)) * (NS * 16)
    EP = -(-E // (NW * C)) * (NW * C)

    src = edge_index[0]
    dst = edge_index[1]
    if EP != E:
        pad_s = jnp.zeros((EP - E,), jnp.int32)
        pad_d = jnp.full((EP - E,), N, jnp.int32)
        src = jnp.concatenate([src, pad_s])
        dst = jnp.concatenate([dst, pad_d])
    xp = jnp.pad(x, ((0, NPAD - N), (0, 0)))

    w3a = W3[:, :H]
    w3b = W3[:, H:]
    b1r = b1.reshape(1, H)
    b2r = b2.reshape(1, H)
    b3r = b3.reshape(1, H)
    w4 = W4.reshape(H)

    hist = _make_hist(NPAD, EP)
    scatter = _make_scatter(NPAD, EP)
    edge_score = _make_edge_score(NPAD, EP)

    deg_parts = hist(dst)

    grid = NPAD // RB
    dp_s = pl.BlockSpec((NC, RB, H), lambda i: (0, i, 0))
    acc_s = pl.BlockSpec((NC, RB, H), lambda i: (0, i, 0))
    rows_s = pl.BlockSpec((RB, H), lambda i: (i, 0))
    w_s = pl.BlockSpec((H, H), lambda i: (0, 0))
    b_s = pl.BlockSpec((1, H), lambda i: (0, 0))

    ht1 = pl.pallas_call(
        _tc1_body,
        grid=(grid,),
        in_specs=[dp_s, rows_s, w_s],
        out_specs=rows_s,
        out_shape=jax.ShapeDtypeStruct((NPAD, H), F32),
    )(deg_parts, xp, W1)

    acc1 = scatter(ht1, src, dst)

    ht2 = pl.pallas_call(
        _tc2_body,
        grid=(grid,),
        in_specs=[dp_s, acc_s, rows_s, w_s, b_s],
        out_specs=rows_s,
        out_shape=jax.ShapeDtypeStruct((NPAD, H), F32),
    )(deg_parts, acc1, ht1, W2, b1r)

    acc2 = scatter(ht2, src, dst)

    A, B = pl.pallas_call(
        _tc3_body,
        grid=(grid,),
        in_specs=[dp_s, acc_s, rows_s, w_s, w_s, b_s, b_s],
        out_specs=[rows_s, rows_s],
        out_shape=[
            jax.ShapeDtypeStruct((NPAD, H), F32),
            jax.ShapeDtypeStruct((NPAD, H), F32),
        ],
    )(deg_parts, acc2, ht2, w3a, w3b, b2r, b3r)

    s16 = edge_score(A, B, src, dst, w4)

    GB = 16
    out = pl.pallas_call(
        _tc4_body,
        grid=(EP // 128 // GB,),
        in_specs=[
            pl.BlockSpec((GB, 128, 16), lambda i: (i, 0, 0)),
            pl.BlockSpec(memory_space=pltpu.SMEM),
        ],
        out_specs=pl.BlockSpec((GB, 128), lambda i: (i, 0)),
        out_shape=jax.ShapeDtypeStruct((EP // 128, 128), F32),
    )(s16.reshape(EP // 128, 128, 16), b4)

    return out.reshape(EP)[:E]
